# baseline (speedup 1.0000x reference)
import jax
import jax.numpy as jnp
from jax.experimental import pallas as pl
from jax.experimental.pallas import tpu as pltpu

N = 10000
H = 128
NCLASSES = 40
BM = 400
NSPLIT = 5
HB = BM // NSPLIT
NBUF = 2
NI = N // BM
NSTEP = 2 * NI


def _half_copy(adj_hbm, buf, sem, g, half):
    rows = (g % NI) * BM + half * HB
    return pltpu.make_async_copy(
        adj_hbm.at[pl.ds(rows, HB), :],
        buf.at[g % NBUF, pl.ds(half * HB, HB), :],
        sem.at[g % NBUF, half])


def _start_block(adj_hbm, buf, sem, g):
    for k in range(NSPLIT):
        _half_copy(adj_hbm, buf, sem, g, k).start()


def _mega_kernel(feat_ref, w1_ref, b1_ref, adj_hbm, wih_ref, whh_ref,
                 bih_ref, bhh_ref, wg1_ref, bg1_ref, wg2_ref, bg2_ref,
                 wc_ref, bc_ref, wd_ref, bd_ref, pred_ref,
                 xbuf, buf, sem):
    g = pl.program_id(0)
    l = g // NI
    i = g % NI

    @pl.when(g == 0)
    def _():
        _start_block(adj_hbm, buf, sem, 0)

        def _x0_slab(j, carry):
            xbuf[0, pl.ds(j * 1000, 1000), :] = jax.nn.relu(
                jnp.dot(feat_ref[pl.ds(j * 1000, 1000), :], w1_ref[...])
                + b1_ref[...])
            return carry

        jax.lax.fori_loop(0, N // 1000, _x0_slab, 0)

    nxt = g + NBUF - 1

    @pl.when(nxt < NSTEP)
    def _():
        _start_block(adj_hbm, buf, sem, nxt)

    for k in range(NSPLIT):
        _half_copy(adj_hbm, buf, sem, g, k).wait()
    a = buf[g % NBUF]

    y = jnp.dot(a, xbuf[l], preferred_element_type=jnp.float32)
    h = xbuf[l, pl.ds(i * BM, BM), :]
    gi = jnp.dot(y, wih_ref[l]) + bih_ref[l]
    gh = jnp.dot(h, whh_ref[l]) + bhh_ref[l]
    r = jax.nn.sigmoid(gi[:, :H] + gh[:, :H])
    z = jax.nn.sigmoid(gi[:, H:2 * H] + gh[:, H:2 * H])
    n = jnp.tanh(gi[:, 2 * H:] + r * gh[:, 2 * H:])
    hn = (1.0 - z) * n + z * h
    out = jax.nn.relu(
        jnp.dot(jax.nn.relu(jnp.dot(hn, wg1_ref[l]) + bg1_ref[l]),
                wg2_ref[l]) + bg2_ref[l])

    @pl.when(l == 0)
    def _():
        xbuf[1, pl.ds(i * BM, BM), :] = out

    @pl.when(l == 1)
    def _():
        t = jnp.dot(jax.nn.relu(jnp.dot(out, wc_ref[...]) + bc_ref[...]),
                    wd_ref[...]) + bd_ref[...]
        m = jnp.max(t, axis=1, keepdims=True)
        e = jnp.exp(t - m)
        pred_ref[pl.ds(i * BM, BM), :] = e / jnp.sum(e, axis=1,
                                                     keepdims=True)


def _resident(shape):
    return pl.BlockSpec(shape, lambda g: (0,) * len(shape),
                        pipeline_mode=pl.Buffered(buffer_count=1))


def kernel(features, adj, W1, b1, Wih, Whh, bih, bhh, Wg1, bg1, Wg2, bg2,
           Wc, bc, Wd, bd):
    wih_t = jnp.transpose(Wih, (0, 2, 1))
    whh_t = jnp.transpose(Whh, (0, 2, 1))
    return pl.pallas_call(
        _mega_kernel,
        grid=(NSTEP,),
        in_specs=[
            _resident((N, H)),
            _resident((H, H)),
            _resident((1, H)),
            pl.BlockSpec(memory_space=pl.ANY),
            _resident((2, H, 3 * H)),
            _resident((2, H, 3 * H)),
            _resident((2, 1, 3 * H)),
            _resident((2, 1, 3 * H)),
            _resident((2, H, H)),
            _resident((2, 1, H)),
            _resident((2, H, H)),
            _resident((2, 1, H)),
            _resident((H, H)),
            _resident((1, H)),
            _resident((H, NCLASSES)),
            _resident((1, NCLASSES)),
        ],
        out_specs=pl.BlockSpec((N, NCLASSES), lambda g: (0, 0),
                               pipeline_mode=pl.Buffered(buffer_count=1)),
        out_shape=jax.ShapeDtypeStruct((N, NCLASSES), jnp.float32),
        scratch_shapes=[
            pltpu.VMEM((2, N, H), jnp.float32),
            pltpu.VMEM((NBUF, BM, N), jnp.float32),
            pltpu.SemaphoreType.DMA((NBUF, NSPLIT)),
        ],
        compiler_params=pltpu.CompilerParams(
            dimension_semantics=("arbitrary",),
            vmem_limit_bytes=64 * 1024 * 1024),
    )(features, W1, b1.reshape(1, H), adj, wih_t, whh_t,
      bih.reshape(2, 1, 3 * H), bhh.reshape(2, 1, 3 * H),
      Wg1, bg1.reshape(2, 1, H), Wg2, bg2.reshape(2, 1, H),
      Wc, bc.reshape(1, H), Wd, bd.reshape(1, NCLASSES))


# speedup vs baseline: 1.0125x; 1.0125x over previous
import jax
import jax.numpy as jnp
from jax.experimental import pallas as pl
from jax.experimental.pallas import tpu as pltpu

N = 10000
H = 128
NCLASSES = 40
BM = 400
NSPLIT = 5
HB = BM // NSPLIT
NBUF = 2
NI = N // BM
NSTEP = 2 * NI


def _half_copy(adj_hbm, buf, sem, g, half):
    rows = (g % NI) * BM + half * HB
    return pltpu.make_async_copy(
        adj_hbm.at[pl.ds(rows, HB), :],
        buf.at[g % NBUF, pl.ds(half * HB, HB), :],
        sem.at[g % NBUF, half])


def _start_block(adj_hbm, buf, sem, g):
    for k in range(NSPLIT):
        _half_copy(adj_hbm, buf, sem, g, k).start()


def _mega_kernel(feat_ref, w1_ref, b1_ref, adj_hbm, wih_ref, whh_ref,
                 bih_ref, bhh_ref, wg1_ref, bg1_ref, wg2_ref, bg2_ref,
                 wc_ref, bc_ref, wd_ref, bd_ref, pred_ref,
                 xbuf, buf, sem):
    g = pl.program_id(0)
    l = g // NI
    i = g % NI

    @pl.when(g == 0)
    def _():
        _start_block(adj_hbm, buf, sem, 0)

        def _x0_slab(j, carry):
            xbuf[0, pl.ds(j * 1000, 1000), :] = jax.nn.relu(
                jnp.dot(feat_ref[pl.ds(j * 1000, 1000), :], w1_ref[...])
                + b1_ref[...])
            return carry

        jax.lax.fori_loop(0, N // 1000, _x0_slab, 0)

    nxt = g + NBUF - 1

    @pl.when(nxt < NSTEP)
    def _():
        _start_block(adj_hbm, buf, sem, nxt)

    parts = []
    for k in range(NSPLIT):
        _half_copy(adj_hbm, buf, sem, g, k).wait()
        parts.append(jnp.dot(buf[g % NBUF, pl.ds(k * HB, HB), :], xbuf[l],
                             preferred_element_type=jnp.float32))
    y = jnp.concatenate(parts, axis=0)
    h = xbuf[l, pl.ds(i * BM, BM), :]
    gi = jnp.dot(y, wih_ref[l]) + bih_ref[l]
    gh = jnp.dot(h, whh_ref[l]) + bhh_ref[l]
    r = jax.nn.sigmoid(gi[:, :H] + gh[:, :H])
    z = jax.nn.sigmoid(gi[:, H:2 * H] + gh[:, H:2 * H])
    n = jnp.tanh(gi[:, 2 * H:] + r * gh[:, 2 * H:])
    hn = (1.0 - z) * n + z * h
    out = jax.nn.relu(
        jnp.dot(jax.nn.relu(jnp.dot(hn, wg1_ref[l]) + bg1_ref[l]),
                wg2_ref[l]) + bg2_ref[l])

    @pl.when(l == 0)
    def _():
        xbuf[1, pl.ds(i * BM, BM), :] = out

    @pl.when(l == 1)
    def _():
        t = jnp.dot(jax.nn.relu(jnp.dot(out, wc_ref[...]) + bc_ref[...]),
                    wd_ref[...]) + bd_ref[...]
        m = jnp.max(t, axis=1, keepdims=True)
        e = jnp.exp(t - m)
        pred_ref[pl.ds(i * BM, BM), :] = e / jnp.sum(e, axis=1,
                                                     keepdims=True)


def _resident(shape):
    return pl.BlockSpec(shape, lambda g: (0,) * len(shape),
                        pipeline_mode=pl.Buffered(buffer_count=1))


def kernel(features, adj, W1, b1, Wih, Whh, bih, bhh, Wg1, bg1, Wg2, bg2,
           Wc, bc, Wd, bd):
    wih_t = jnp.transpose(Wih, (0, 2, 1))
    whh_t = jnp.transpose(Whh, (0, 2, 1))
    return pl.pallas_call(
        _mega_kernel,
        grid=(NSTEP,),
        in_specs=[
            _resident((N, H)),
            _resident((H, H)),
            _resident((1, H)),
            pl.BlockSpec(memory_space=pl.ANY),
            _resident((2, H, 3 * H)),
            _resident((2, H, 3 * H)),
            _resident((2, 1, 3 * H)),
            _resident((2, 1, 3 * H)),
            _resident((2, H, H)),
            _resident((2, 1, H)),
            _resident((2, H, H)),
            _resident((2, 1, H)),
            _resident((H, H)),
            _resident((1, H)),
            _resident((H, NCLASSES)),
            _resident((1, NCLASSES)),
        ],
        out_specs=pl.BlockSpec((N, NCLASSES), lambda g: (0, 0),
                               pipeline_mode=pl.Buffered(buffer_count=1)),
        out_shape=jax.ShapeDtypeStruct((N, NCLASSES), jnp.float32),
        scratch_shapes=[
            pltpu.VMEM((2, N, H), jnp.float32),
            pltpu.VMEM((NBUF, BM, N), jnp.float32),
            pltpu.SemaphoreType.DMA((NBUF, NSPLIT)),
        ],
        compiler_params=pltpu.CompilerParams(
            dimension_semantics=("arbitrary",),
            vmem_limit_bytes=64 * 1024 * 1024),
    )(features, W1, b1.reshape(1, H), adj, wih_t, whh_t,
      bih.reshape(2, 1, 3 * H), bhh.reshape(2, 1, 3 * H),
      Wg1, bg1.reshape(2, 1, H), Wg2, bg2.reshape(2, 1, H),
      Wc, bc.reshape(1, H), Wd, bd.reshape(1, NCLASSES))
